# Initial kernel scaffold; baseline (speedup 1.0000x reference)
import jax
import jax.numpy as jnp
from jax.experimental import pallas as pl


def kernel(x, edge_index, relation_type, bases_x, comp_x, root_x, bias_x, bases_h, comp_h, root_h, bias_h, Wp, bp):
    raise NotImplementedError("write your pallas kernel here")


# speedup vs baseline: 4.4635x; 4.4635x over previous
import functools
import jax
import jax.numpy as jnp
from jax import lax
from jax.experimental import pallas as pl
from jax.experimental.pallas import tpu as pltpu
from jax.experimental.pallas import tpu_sc as plsc

N = 10000
E = 320000
R = 8
D = 128

NC = 2
NS = 16
NPASS = 13
DP = 416
ROWS = DP * R
TRASH = ROWS
SROWS = ROWS + 16
CHK = 80
EPT = E // NS
NCHUNK = EPT // CHK
RPT = ROWS // NS
DRN = 208
OUT_ROWS = NPASS * NC * ROWS


def _sc_body(src_hbm, dst_hbm, rel_hbm, x_hbm, out_hbm,
             src_v, dst_v, rel_v, lb_v, rows_v, ones_v, dbuf, ddeg,
             sums_sh, deg_sh, sem):
    cid = lax.axis_index("c")
    sid = lax.axis_index("s")

    zero16 = jnp.zeros((16,), jnp.float32)
    one16 = jnp.ones((16,), jnp.float32)

    def init_ones(i, c):
        for j in range(D // 16):
            ones_v[i, pl.ds(j * 16, 16)] = one16
        return c
    lax.fori_loop(0, CHK, init_ones, 0)

    for p in range(NPASS):
        base = (p * NC + cid) * ROWS

        def init_zero(i, c):
            for j in range(D // 16):
                dbuf[i, pl.ds(j * 16, 16)] = zero16
            return c
        lax.fori_loop(0, DRN, init_zero, 0)
        for k in range(RPT // DRN):
            r0 = sid * RPT + k * DRN
            pltpu.sync_copy(dbuf, sums_sh.at[pl.ds(r0, DRN)])
            pltpu.sync_copy(dbuf, deg_sh.at[pl.ds(r0, DRN)])
        plsc.subcore_barrier()

        def chunk(t, c):
            e0 = sid * EPT + t * CHK
            pltpu.sync_copy(src_hbm.at[pl.ds(e0, CHK)], src_v)
            pltpu.sync_copy(dst_hbm.at[pl.ds(e0, CHK)], dst_v)
            pltpu.sync_copy(rel_hbm.at[pl.ds(e0, CHK)], rel_v)
            for g in range(CHK // 16):
                d = dst_v[pl.ds(g * 16, 16)]
                r = rel_v[pl.ds(g * 16, 16)]
                row = d * R + r - base
                oor = (row < 0) | (row >= ROWS)
                lb_v[pl.ds(g * 16, 16)] = jnp.where(oor, TRASH, row)
            pltpu.async_copy(x_hbm.at[src_v], rows_v, sem).wait()
            pltpu.sync_copy(rows_v, sums_sh.at[lb_v], add=True)
            pltpu.sync_copy(ones_v, deg_sh.at[lb_v], add=True)
            return c
        lax.fori_loop(0, NCHUNK, chunk, 0)
        plsc.subcore_barrier()

        for k in range(RPT // DRN):
            r0 = sid * RPT + k * DRN
            pltpu.sync_copy(sums_sh.at[pl.ds(r0, DRN)], dbuf)
            pltpu.sync_copy(deg_sh.at[pl.ds(r0, DRN)], ddeg)

            def scale_row(i, c2):
                dv = ddeg[i, pl.ds(0, 16)]
                sc = jnp.where(dv > 0.0, 1.0 / dv, 0.0)
                for j in range(D // 16):
                    s = pl.ds(j * 16, 16)
                    dbuf[i, s] = dbuf[i, s] * sc
                return c2
            lax.fori_loop(0, DRN, scale_row, 0)
            pltpu.sync_copy(dbuf, out_hbm.at[pl.ds(base + r0, DRN)])
        plsc.subcore_barrier()


_sc_seg_mean = functools.partial(
    pl.kernel,
    out_type=jax.ShapeDtypeStruct((OUT_ROWS, D), jnp.float32),
    mesh=plsc.VectorSubcoreMesh(core_axis_name="c", subcore_axis_name="s"),
    scratch_types=[
        pltpu.VMEM((CHK,), jnp.int32),
        pltpu.VMEM((CHK,), jnp.int32),
        pltpu.VMEM((CHK,), jnp.int32),
        pltpu.VMEM((CHK,), jnp.int32),
        pltpu.VMEM((CHK, D), jnp.float32),
        pltpu.VMEM((CHK, D), jnp.float32),
        pltpu.VMEM((DRN, D), jnp.float32),
        pltpu.VMEM((DRN, D), jnp.float32),
        pltpu.VMEM_SHARED((SROWS, D), jnp.float32),
        pltpu.VMEM_SHARED((SROWS, D), jnp.float32),
        pltpu.SemaphoreType.DMA,
    ],
)(_sc_body)


BN = 400
G3 = 3 * D


def _tc_body(m_ref, x_ref, wm_ref, wr_ref, wp_ref, b_ref, bp_ref,
             out_ref, h_ref, c_ref):
    g = jnp.dot(m_ref[...], wm_ref[...], preferred_element_type=jnp.float32)
    g = g + jnp.dot(x_ref[...], wr_ref[...], preferred_element_type=jnp.float32)
    g = g + b_ref[...]
    ig = jax.nn.sigmoid(g[:, 0:D])
    ct = jnp.tanh(g[:, D:2 * D])
    og = jax.nn.sigmoid(g[:, 2 * D:3 * D])
    c = ig * ct
    h = og * jnp.tanh(c)
    c_ref[...] = c
    h_ref[...] = h
    out_ref[...] = jnp.dot(jnp.maximum(h, 0.0), wp_ref[...],
                           preferred_element_type=jnp.float32) + bp_ref[...]


def _tc_call(mflat, x, wm, wr, wp, bvec, bpvec):
    return pl.pallas_call(
        _tc_body,
        grid=(N // BN,),
        in_specs=[
            pl.BlockSpec((BN, R * D), lambda i: (i, 0)),
            pl.BlockSpec((BN, D), lambda i: (i, 0)),
            pl.BlockSpec((R * D, G3), lambda i: (0, 0)),
            pl.BlockSpec((D, G3), lambda i: (0, 0)),
            pl.BlockSpec((D, D), lambda i: (0, 0)),
            pl.BlockSpec((1, G3), lambda i: (0, 0)),
            pl.BlockSpec((1, D), lambda i: (0, 0)),
        ],
        out_specs=[
            pl.BlockSpec((BN, D), lambda i: (i, 0)),
            pl.BlockSpec((BN, D), lambda i: (i, 0)),
            pl.BlockSpec((BN, D), lambda i: (i, 0)),
        ],
        out_shape=[
            jax.ShapeDtypeStruct((N, D), jnp.float32),
            jax.ShapeDtypeStruct((N, D), jnp.float32),
            jax.ShapeDtypeStruct((N, D), jnp.float32),
        ],
    )(mflat, x, wm, wr, wp, bvec, bpvec)


def kernel(x, edge_index, relation_type, bases_x, comp_x, root_x, bias_x,
           bases_h, comp_h, root_h, bias_h, Wp, bp):
    src = edge_index[0]
    dst = edge_index[1]

    means = _sc_seg_mean(src, dst, relation_type, x)
    mflat = means[:N * R].reshape(N, R * D)

    gsel = jnp.array([0, 2, 3])
    wg = jnp.einsum('grb,gbio->grio', comp_x[gsel], bases_x[gsel])
    wm = wg.transpose(1, 2, 0, 3).reshape(R * D, G3)
    wr = root_x[gsel].transpose(1, 0, 2).reshape(D, G3)
    bvec = (bias_x + bias_h)[gsel].reshape(1, G3)

    out, h, c = _tc_call(mflat, x, wm, wr, Wp, bvec, bp.reshape(1, D))
    return (out, jnp.stack((h, c), axis=0))
